# baseline (speedup 1.0000x reference)
import functools

import jax
import jax.numpy as jnp
from jax import lax
from jax.experimental import pallas as pl
from jax.experimental.pallas import tpu as pltpu
from jax.experimental.pallas import tpu_sc as plsc

_RNG = 7
_SIDE = _RNG + 1
_PAD_IDX = _SIDE * _SIDE


@functools.lru_cache(maxsize=None)
def _build_sc_fn(H, nx, ny):
    N = nx * ny
    NC, NS = 2, 16
    NW = NC * NS
    NB = NW // H
    BR = ny // NB
    NP = 128 // ny
    FLEN = (2 * _RNG + 1) * ny
    ZPRE = ny * (nx - 1 - _RNG)
    ZP = 2048

    mesh = plsc.VectorSubcoreMesh(core_axis_name="c", subcore_axis_name="s")

    @functools.partial(
        pl.kernel,
        mesh=mesh,
        out_type=jax.ShapeDtypeStruct((H, N, N), jnp.float32),
        scratch_types=[
            pltpu.VMEM((_PAD_IDX + 1, H), jnp.float32),
        ]
        + [pltpu.VMEM((BR, ZP), jnp.float32) for _ in range(NP)]
        + [pltpu.SemaphoreType.DMA],
        compiler_params=pltpu.CompilerParams(
            needs_layout_passes=False, skip_device_barrier=True
        ),
    )
    def sc(table_hbm, out_hbm, tab_v, z0, z1, z2, z3, sem):
        zp = (z0, z1, z2, z3)
        cid = lax.axis_index("c")
        sid = lax.axis_index("s")
        wid = sid * NC + cid
        h = wid // NB
        blk = wid % NB
        lane = lax.iota(jnp.int32, 16)

        pltpu.sync_copy(table_hbm, tab_v)

        ZMAX = ZPRE + (_RNG - NP + 1) * ny + N
        zeros16 = jnp.zeros((16,), jnp.float32)
        for p in range(NP):
            front = ZPRE - p * ny
            back = front + FLEN

            def zfront(j, carry, front=front):
                for r in range(BR):
                    zp[p][r, pl.ds(j * 16, 16)] = zeros16
                return carry

            def zback(j, carry, back=back):
                for r in range(BR):
                    zp[p][r, pl.ds(back + j * 16, 16)] = zeros16
                return carry

            lax.fori_loop(0, front // 16, zfront, 0)
            lax.fori_loop(0, (ZMAX - back) // 16, zback, 0)

        hvec = jnp.zeros((16,), jnp.int32) + h
        for r in range(BR):
            yi = blk * BR + r

            def f_body(v, carry):
                tt = lane + v * 16
                a = jnp.abs((tt >> 5) - _RNG)
                b = jnp.abs(yi - (tt & (ny - 1)))
                tok = jnp.where(b <= _RNG, a * _SIDE + b, _PAD_IDX)
                vals = plsc.load_gather(tab_v, [tok, hvec])
                for p in range(NP):
                    zp[p][r, pl.ds(ZPRE - p * ny + v * 16, 16)] = vals
                return carry

            lax.fori_loop(0, FLEN // 16, f_body, 0)

        def out_body(g, carry):
            off = pl.multiple_of(ZPRE + _RNG * ny - NP * ny * g - 3 * ny, 128)
            for p in range(NP):
                xi = g * NP + (NP - 1 - p)
                row0 = pl.multiple_of(xi * ny + blk * BR, BR)
                pltpu.async_copy(
                    zp[p].at[:, pl.ds(off, N)],
                    out_hbm.at[h, pl.ds(row0, BR)],
                    sem,
                )
            return carry

        lax.fori_loop(0, nx // NP, out_body, 0)

        def drain_body(j, carry):
            pltpu.make_async_copy(
                z0.at[:, pl.ds(0, N)], out_hbm.at[0, pl.ds(0, BR)], sem
            ).wait()
            return carry

        lax.fori_loop(0, nx, drain_body, 0)

    return sc


def kernel(x, table):
    nx, ny = x.shape[-2], x.shape[-1]
    H = table.shape[1]
    fn = _build_sc_fn(H, nx, ny)
    return fn(table)


# speedup vs baseline: 1.0706x; 1.0706x over previous
import functools

import jax
import jax.numpy as jnp
from jax import lax
from jax.experimental import pallas as pl
from jax.experimental.pallas import tpu as pltpu
from jax.experimental.pallas import tpu_sc as plsc

_RNG = 7
_SIDE = _RNG + 1
_PAD_IDX = _SIDE * _SIDE


@functools.lru_cache(maxsize=None)
def _build_sc_fn(H, nx, ny):
    N = nx * ny
    NC, NS = 2, 16
    NW = NC * NS
    NB = NW // H
    BR = ny // NB
    NP = 128 // ny
    FLEN = (2 * _RNG + 1) * ny
    ZPRE = ny * (nx - 1 - _RNG)
    ZP = 2048

    mesh = plsc.VectorSubcoreMesh(core_axis_name="c", subcore_axis_name="s")

    @functools.partial(
        pl.kernel,
        mesh=mesh,
        out_type=jax.ShapeDtypeStruct((H, N, N), jnp.float32),
        scratch_types=[
            pltpu.VMEM(((_PAD_IDX + 1) * H,), jnp.float32),
        ]
        + [pltpu.VMEM((BR, ZP), jnp.float32) for _ in range(NP)]
        + [pltpu.SemaphoreType.DMA],
        compiler_params=pltpu.CompilerParams(
            needs_layout_passes=False, skip_device_barrier=True
        ),
    )
    def sc(table_hbm, out_hbm, tab_v, z0, z1, z2, z3, sem):
        zp = (z0, z1, z2, z3)
        cid = lax.axis_index("c")
        sid = lax.axis_index("s")
        wid = sid * NC + cid
        h = wid // NB
        blk = wid % NB
        lane = lax.iota(jnp.int32, 16)

        pltpu.sync_copy(table_hbm, tab_v)

        ZMAX = ZPRE + (_RNG - NP + 1) * ny + N
        zeros16 = jnp.zeros((16,), jnp.float32)
        for p in range(NP):
            front = ZPRE - p * ny
            back = front + FLEN

            def zfront(j, carry, front=front):
                for r in range(BR):
                    zp[p][r, pl.ds(j * 16, 16)] = zeros16
                return carry

            def zback(j, carry, back=back):
                for r in range(BR):
                    zp[p][r, pl.ds(back + j * 16, 16)] = zeros16
                return carry

            lax.fori_loop(0, front // 16, zfront, 0)
            lax.fori_loop(0, (ZMAX - back) // 16, zback, 0)

        hvec = jnp.zeros((16,), jnp.int32) + h
        for r in range(BR):
            yi = blk * BR + r

            def f_body(v, carry):
                tt = lane + v * 16
                a = jnp.abs((tt >> 5) - _RNG)
                b = jnp.abs(yi - (tt & (ny - 1)))
                tok = jnp.where(b <= _RNG, a * _SIDE + b, _PAD_IDX)
                vals = plsc.load_gather(tab_v, [tok * H + hvec])
                for p in range(NP):
                    zp[p][r, pl.ds(ZPRE - p * ny + v * 16, 16)] = vals
                return carry

            lax.fori_loop(0, FLEN // 16, f_body, 0)

        def out_body(g, carry):
            off = pl.multiple_of(ZPRE + _RNG * ny - NP * ny * g - 3 * ny, 128)
            for p in range(NP):
                xi = g * NP + (NP - 1 - p)
                row0 = pl.multiple_of(xi * ny + blk * BR, BR)
                pltpu.async_copy(
                    zp[p].at[:, pl.ds(off, N)],
                    out_hbm.at[h, pl.ds(row0, BR)],
                    sem,
                )
            return carry

        lax.fori_loop(0, nx // NP, out_body, 0)

        def drain_body(j, carry):
            pltpu.make_async_copy(
                z0.at[:, pl.ds(0, N)], out_hbm.at[0, pl.ds(0, BR)], sem
            ).wait()
            return carry

        lax.fori_loop(0, nx, drain_body, 0)

    return sc


def kernel(x, table):
    nx, ny = x.shape[-2], x.shape[-1]
    H = table.shape[1]
    fn = _build_sc_fn(H, nx, ny)
    return fn(table.reshape(-1))
